# baseline (speedup 1.0000x reference)
import functools

import jax
import jax.numpy as jnp
from jax import lax
from jax.experimental import pallas as pl
from jax.experimental.pallas import tpu as pltpu
from jax.experimental.pallas import tpu_sc as plsc

B = 4096
S = 200
D = 64
DP = 128
NUM_LANG = 10

NC = 2
NS = 16
NW = NC * NS
SPW = B // NW
HALF = S // 2
HPW = 2 * SPW
NLANE = 16


def _sc_embedding_bag(table2, idx2d):
  mesh = plsc.VectorSubcoreMesh(core_axis_name="c", subcore_axis_name="s")

  @functools.partial(
      pl.kernel,
      out_type=jax.ShapeDtypeStruct((B, D), jnp.float32),
      mesh=mesh,
      scratch_types=[
          pltpu.VMEM((HPW, HALF), jnp.int32),
          pltpu.VMEM((S, D), jnp.float32),
          pltpu.VMEM((S, D), jnp.float32),
          pltpu.VMEM((SPW, D), jnp.float32),
          pltpu.SemaphoreType.DMA,
          pltpu.SemaphoreType.DMA,
      ],
      compiler_params=pltpu.CompilerParams(use_tc_tiling_on_sc=False),
  )
  def sc_kernel(table_hbm, idx_hbm, out_hbm, idx_v, buf0, buf1, acc_v, sem0,
                sem1):
    wid = lax.axis_index("s") * NC + lax.axis_index("c")
    pltpu.sync_copy(idx_hbm.at[pl.ds(wid * HPW, HPW)], idx_v)

    zero = jnp.zeros((NLANE,), jnp.float32)

    def issue(i, buf, sem):
      r0 = jnp.minimum(2 * i, HPW - 2)
      pltpu.async_copy(table_hbm.at[idx_v.at[r0]], buf.at[pl.ds(0, HALF)], sem)
      pltpu.async_copy(
          table_hbm.at[idx_v.at[r0 + 1]], buf.at[pl.ds(HALF, HALF)], sem)

    def wait(buf, sem):
      pltpu.make_async_copy(table_hbm.at[idx_v.at[0]],
                            buf.at[pl.ds(0, HALF)], sem).wait()
      pltpu.make_async_copy(table_hbm.at[idx_v.at[0]],
                            buf.at[pl.ds(HALF, HALF)], sem).wait()

    def reduce_into(buf, i):
      def red(r, accs):
        return tuple(
            accs[k] + buf[r, pl.ds(k * NLANE, NLANE)] for k in range(4))

      accs = lax.fori_loop(0, S, red, (zero, zero, zero, zero), unroll=10)
      for k in range(4):
        acc_v[i, pl.ds(k * NLANE, NLANE)] = accs[k]

    issue(0, buf0, sem0)
    issue(1, buf1, sem1)

    @pl.loop(0, SPW, step=2)
    def _(i):
      wait(buf0, sem0)
      reduce_into(buf0, i)
      issue(i + 2, buf0, sem0)
      wait(buf1, sem1)
      reduce_into(buf1, i + 1)
      issue(i + 3, buf1, sem1)

    wait(buf0, sem0)
    wait(buf1, sem1)
    pltpu.sync_copy(acc_v, out_hbm.at[pl.ds(wid * SPW, SPW)])

  return sc_kernel(table2, idx2d)


def _tc_relayout(tableT):
  V = tableT.shape[1]
  VB = 2048

  def body(x_ref, e_ref, o_ref):
    o_ref[:, 0:D] = lax.dot_general(
        x_ref[...], e_ref[...], (((0,), (0,)), ((), ())),
        precision=lax.Precision.HIGHEST,
        preferred_element_type=jnp.float32)

  eye = jnp.eye(D, dtype=jnp.float32)
  return pl.pallas_call(
      body,
      grid=(pl.cdiv(V, VB),),
      in_specs=[pl.BlockSpec((D, VB), lambda i: (0, i)),
                pl.BlockSpec((D, D), lambda i: (0, 0))],
      out_specs=pl.BlockSpec((VB, DP), lambda i: (i, 0)),
      out_shape=jax.ShapeDtypeStruct((V, DP), jnp.float32),
  )(tableT, eye)


def _tc_head(sums, W, b):
  def body(x_ref, w_ref, b_ref, o_ref):
    acc = jnp.dot(x_ref[...], w_ref[...], preferred_element_type=jnp.float32)
    o_ref[...] = acc * (1.0 / S) + b_ref[...]

  return pl.pallas_call(
      body,
      out_shape=jax.ShapeDtypeStruct((B, NUM_LANG), jnp.float32),
  )(sums, W, b.reshape(1, NUM_LANG))


@jax.jit
def kernel(inputs, table, W, b):
  idx2d = (2 * inputs.astype(jnp.int32)).reshape(2 * B, HALF)
  table128 = _tc_relayout(table.T)
  table2 = table128.reshape(2 * 1000000, D)
  sums = _sc_embedding_bag(table2, idx2d)
  logits = _tc_head(sums, W, b)
  return {"logits": logits}


# speedup vs baseline: 1.8396x; 1.8396x over previous
import functools

import jax
import jax.numpy as jnp
from jax import lax
from jax.experimental import pallas as pl
from jax.experimental.pallas import tpu as pltpu
from jax.experimental.pallas import tpu_sc as plsc

B = 4096
S = 200
D = 64
DP = 128
NUM_LANG = 10

NC = 2
NS = 16
NW = NC * NS
SPW = B // NW
HALF = S // 2
HPW = 2 * SPW
NLANE = 16


def _sc_embedding_bag(table2, idx2d):
  mesh = plsc.VectorSubcoreMesh(core_axis_name="c", subcore_axis_name="s")

  @functools.partial(
      pl.kernel,
      out_type=jax.ShapeDtypeStruct((B, D), jnp.float32),
      mesh=mesh,
      scratch_types=[
          pltpu.VMEM((HPW, HALF), jnp.int32),
          pltpu.VMEM((S, D), jnp.float32),
          pltpu.VMEM((S, D), jnp.float32),
          pltpu.VMEM((SPW, D), jnp.float32),
          pltpu.SemaphoreType.DMA,
          pltpu.SemaphoreType.DMA,
      ],
      compiler_params=pltpu.CompilerParams(use_tc_tiling_on_sc=False),
  )
  def sc_kernel(table_hbm, idx_hbm, out_hbm, idx_v, buf0, buf1, acc_v, sem0,
                sem1):
    wid = lax.axis_index("s") * NC + lax.axis_index("c")
    pltpu.sync_copy(idx_hbm.at[pl.ds(wid * HPW, HPW)], idx_v)

    zero = jnp.zeros((NLANE,), jnp.float32)

    def issue(i, buf, sem):
      r0 = jnp.minimum(2 * i, HPW - 2)
      pltpu.async_copy(table_hbm.at[idx_v.at[r0]], buf.at[pl.ds(0, HALF)], sem)
      pltpu.async_copy(
          table_hbm.at[idx_v.at[r0 + 1]], buf.at[pl.ds(HALF, HALF)], sem)

    def wait(buf, sem):
      pltpu.make_async_copy(table_hbm.at[idx_v.at[0]],
                            buf.at[pl.ds(0, HALF)], sem).wait()
      pltpu.make_async_copy(table_hbm.at[idx_v.at[0]],
                            buf.at[pl.ds(HALF, HALF)], sem).wait()

    def reduce_into(buf, i):
      def red(r, accs):
        return tuple(
            accs[k] + buf[r, pl.ds(k * NLANE, NLANE)] for k in range(4))

      accs = lax.fori_loop(0, S, red, (zero, zero, zero, zero), unroll=10)
      for k in range(4):
        acc_v[i, pl.ds(k * NLANE, NLANE)] = accs[k]

    issue(0, buf0, sem0)
    issue(1, buf1, sem1)

    @pl.loop(0, SPW, step=2)
    def _(i):
      wait(buf0, sem0)
      reduce_into(buf0, i)
      issue(i + 2, buf0, sem0)
      wait(buf1, sem1)
      reduce_into(buf1, i + 1)
      issue(i + 3, buf1, sem1)

    wait(buf0, sem0)
    wait(buf1, sem1)
    pltpu.sync_copy(acc_v, out_hbm.at[pl.ds(wid * SPW, SPW)])

  return sc_kernel(table2, idx2d)


def _tc_relayout(tableT):
  V = tableT.shape[1]
  VB = 8192

  def body(x_ref, o_ref):
    o_ref[:, 0:D] = x_ref[...].T

  return pl.pallas_call(
      body,
      grid=(pl.cdiv(V, VB),),
      in_specs=[pl.BlockSpec((D, VB), lambda i: (0, i))],
      out_specs=pl.BlockSpec((VB, DP), lambda i: (i, 0)),
      out_shape=jax.ShapeDtypeStruct((V, DP), jnp.float32),
  )(tableT)


def _tc_head(sums, W, b):
  def body(x_ref, w_ref, b_ref, o_ref):
    acc = jnp.dot(x_ref[...], w_ref[...], preferred_element_type=jnp.float32)
    o_ref[...] = acc * (1.0 / S) + b_ref[...]

  return pl.pallas_call(
      body,
      out_shape=jax.ShapeDtypeStruct((B, NUM_LANG), jnp.float32),
  )(sums, W, b.reshape(1, NUM_LANG))


@jax.jit
def kernel(inputs, table, W, b):
  idx2d = (2 * inputs.astype(jnp.int32)).reshape(2 * B, HALF)
  table128 = _tc_relayout(table.T)
  table2 = table128.reshape(2 * 1000000, D)
  sums = _sc_embedding_bag(table2, idx2d)
  logits = _tc_head(sums, W, b)
  return {"logits": logits}


# speedup vs baseline: 1.9355x; 1.0521x over previous
import functools

import jax
import jax.numpy as jnp
from jax import lax
from jax.experimental import pallas as pl
from jax.experimental.pallas import tpu as pltpu
from jax.experimental.pallas import tpu_sc as plsc

B = 4096
S = 200
D = 64
DP = 128
NUM_LANG = 10

NC = 2
NS = 16
NW = NC * NS
SPW = B // NW
HALF = S // 2
HPW = 2 * SPW
NLANE = 16


def _sc_embedding_bag(table2, idx2d):
  mesh = plsc.VectorSubcoreMesh(core_axis_name="c", subcore_axis_name="s")

  @functools.partial(
      pl.kernel,
      out_type=jax.ShapeDtypeStruct((B, D), jnp.float32),
      mesh=mesh,
      scratch_types=[
          pltpu.VMEM((HPW, HALF), jnp.int32),
          pltpu.VMEM((S, D), jnp.float32),
          pltpu.VMEM((S, D), jnp.float32),
          pltpu.VMEM((SPW, D), jnp.float32),
          pltpu.SemaphoreType.DMA,
          pltpu.SemaphoreType.DMA,
      ],
      compiler_params=pltpu.CompilerParams(use_tc_tiling_on_sc=False),
  )
  def sc_kernel(table_hbm, idx_hbm, out_hbm, idx_v, buf0, buf1, acc_v, sem0,
                sem1):
    wid = lax.axis_index("s") * NC + lax.axis_index("c")
    pltpu.sync_copy(idx_hbm.at[pl.ds(wid * HPW, HPW)], idx_v)

    zero = jnp.zeros((NLANE,), jnp.float32)

    def issue(i, buf, sem):
      r0 = jnp.minimum(2 * i, HPW - 2)
      pltpu.async_copy(table_hbm.at[idx_v.at[r0]], buf.at[pl.ds(0, HALF)], sem)
      pltpu.async_copy(
          table_hbm.at[idx_v.at[r0 + 1]], buf.at[pl.ds(HALF, HALF)], sem)

    def wait(buf, sem):
      pltpu.make_async_copy(table_hbm.at[idx_v.at[0]],
                            buf.at[pl.ds(0, HALF)], sem).wait()
      pltpu.make_async_copy(table_hbm.at[idx_v.at[0]],
                            buf.at[pl.ds(HALF, HALF)], sem).wait()

    def reduce_into(buf, i):
      def red(r, accs):
        return tuple(
            accs[k] + buf[r, pl.ds(k * NLANE, NLANE)] for k in range(4))

      accs = lax.fori_loop(0, S, red, (zero, zero, zero, zero), unroll=10)
      for k in range(4):
        acc_v[i, pl.ds(k * NLANE, NLANE)] = accs[k]

    issue(0, buf0, sem0)
    issue(1, buf1, sem1)

    @pl.loop(0, SPW, step=2)
    def _(i):
      wait(buf0, sem0)
      reduce_into(buf0, i)
      issue(i + 2, buf0, sem0)
      wait(buf1, sem1)
      reduce_into(buf1, i + 1)
      issue(i + 3, buf1, sem1)

    wait(buf0, sem0)
    wait(buf1, sem1)
    pltpu.sync_copy(acc_v, out_hbm.at[pl.ds(wid * SPW, SPW)])

  return sc_kernel(table2, idx2d)


def _tc_relayout(tableT):
  V = tableT.shape[1]
  VB = 16384

  def body(x_ref, o_ref):
    o_ref[:, 0:D] = x_ref[...].T

  return pl.pallas_call(
      body,
      grid=(pl.cdiv(V, VB),),
      in_specs=[pl.BlockSpec((D, VB), lambda i: (0, i))],
      out_specs=pl.BlockSpec((VB, DP), lambda i: (i, 0)),
      out_shape=jax.ShapeDtypeStruct((V, DP), jnp.float32),
  )(tableT)


def _tc_head(sums, W, b):
  def body(x_ref, w_ref, b_ref, o_ref):
    acc = jnp.dot(x_ref[...], w_ref[...], preferred_element_type=jnp.float32)
    o_ref[...] = acc * (1.0 / S) + b_ref[...]

  return pl.pallas_call(
      body,
      out_shape=jax.ShapeDtypeStruct((B, NUM_LANG), jnp.float32),
  )(sums, W, b.reshape(1, NUM_LANG))


@jax.jit
def kernel(inputs, table, W, b):
  idx2d = (2 * inputs.astype(jnp.int32)).reshape(2 * B, HALF)
  table128 = _tc_relayout(table.T)
  table2 = table128.reshape(2 * 1000000, D)
  sums = _sc_embedding_bag(table2, idx2d)
  logits = _tc_head(sums, W, b)
  return {"logits": logits}
